# baseline (speedup 1.0000x reference)
import functools

import jax
import jax.numpy as jnp
from jax import lax
from jax.experimental import pallas as pl
from jax.experimental.pallas import tpu as pltpu
from jax.experimental.pallas import tpu_sc as plsc

N = 4096
OBS = 256
FEAT = 1024
CRIT_H = 1024
TERM_H = 512
OPT_H = 1024
NOPT = 8
ACT = 64

BLK = 256
PADN = N + NOPT * BLK
NBLK = PADN // BLK
NBLK_PAD = 32

NW = 32
TOK_W = N // NW
LANES = 16


def _route_body(obs_hbm, opt_hbm,
                obs_s_hbm, dst_hbm, bexp_hbm, bval_hbm,
                optbuf, dstbuf, obsbuf, bexpbuf, bvalbuf, sem):
    nc = 2
    w = lax.axis_index("s") * nc + lax.axis_index("c")

    pltpu.sync_copy(opt_hbm, optbuf)
    pltpu.sync_copy(obs_hbm.at[pl.ds(w * TOK_W, TOK_W)], obsbuf)

    myfirst = w * (TOK_W // LANES)
    zero16 = jnp.zeros((LANES,), jnp.int32)

    def group_step(g, accs):
        a4 = zero16
        for k in range(TOK_W // LANES):
            v = optbuf[pl.ds((g * (TOK_W // LANES) + k) * LANES, LANES)]
            a4 = a4 + (jnp.int32(1) << (v * 4))
        return tuple(accs[e] + ((a4 >> (e * 4)) & 15) for e in range(NOPT))

    snap = lax.fori_loop(0, w, group_step, (zero16,) * NOPT)
    accs = lax.fori_loop(w, NW, group_step, snap)

    tot = [jnp.sum(accs[e]) for e in range(NOPT)]
    prefix = [jnp.sum(snap[e]) for e in range(NOPT)]

    base, pad = [], []
    run = jnp.int32(0)
    for e in range(NOPT):
        base.append(run)
        pe = ((tot[e] + (BLK - 1)) // BLK) * BLK
        pad.append(pe)
        run = run + pe

    start = [base[e] + prefix[e] for e in range(NOPT)]
    for c in range(TOK_W // LANES):
        v = optbuf[pl.ds((myfirst + c) * LANES, LANES)]
        dstv = zero16
        for e in range(NOPT):
            m = v == e
            mi = m.astype(jnp.int32)
            ranks = plsc.cumsum(mi)
            dstv = jnp.where(m, start[e] + ranks - 1, dstv)
            start[e] = start[e] + jnp.sum(mi)
        dstbuf[pl.ds(c * LANES, LANES)] = dstv

    pltpu.sync_copy(dstbuf, dst_hbm.at[w])
    pltpu.async_copy(obsbuf, obs_s_hbm.at[dstbuf], sem).wait()

    @pl.when(w == 0)
    def _():
        for cc in range(NBLK_PAD // LANES):
            pos = (lax.iota(jnp.int32, LANES) + cc * LANES) * BLK
            ev = jnp.zeros((LANES,), jnp.int32)
            vv = jnp.zeros((LANES,), jnp.int32)
            for e in range(NOPT):
                ev = ev + (pos >= base[e] + pad[e]).astype(jnp.int32)
                vv = vv + ((pos >= base[e]) & (pos < base[e] + tot[e])).astype(jnp.int32)
            bexpbuf[pl.ds(cc * LANES, LANES)] = jnp.minimum(ev, NOPT - 1)
            bvalbuf[pl.ds(cc * LANES, LANES)] = vv
        pltpu.sync_copy(bexpbuf, bexp_hbm)
        pltpu.sync_copy(bvalbuf, bval_hbm)


def _make_route(interpret=False):
    return functools.partial(
        pl.kernel,
        out_type=[
            jax.ShapeDtypeStruct((PADN, OBS), jnp.float32),
            jax.ShapeDtypeStruct((NW, TOK_W), jnp.int32),
            jax.ShapeDtypeStruct((NBLK_PAD,), jnp.int32),
            jax.ShapeDtypeStruct((NBLK_PAD,), jnp.int32),
        ],
        mesh=plsc.VectorSubcoreMesh(core_axis_name="c", subcore_axis_name="s", num_cores=2, num_subcores=16),
        scratch_types=[
            pltpu.VMEM((N,), jnp.int32),
            pltpu.VMEM((TOK_W,), jnp.int32),
            pltpu.VMEM((TOK_W, OBS), jnp.float32),
            pltpu.VMEM((NBLK_PAD,), jnp.int32),
            pltpu.VMEM((NBLK_PAD,), jnp.int32),
            pltpu.SemaphoreType.DMA,
        ],
        compiler_params=pltpu.CompilerParams(needs_layout_passes=False),
        interpret=interpret,
    )(_route_body)


def _tc_body(bexp_ref, bval_ref,
             obs_ref, wf0, bf0, wf1, bf1, wc0, bc0, wc1, bc1,
             wt0, bt0, wt1, bt1, we0a, be0a, we1a, be1a,
             we0b_, be0b_, we1b_, be1b_,
             res_ref,
             swf0, swf1, swc0, swc1, swt0, swt1):
    b = pl.program_id(0)
    bf16 = jnp.bfloat16

    @pl.when(b == 0)
    def _():
        swf0[...] = wf0[...].astype(bf16)
        swf1[...] = wf1[...].astype(bf16)
        swc0[...] = wc0[...].astype(bf16)
        swc1[...] = wc1[...].astype(bf16)
        swt0[...] = wt0[...].astype(bf16)
        swt1[...] = wt1[...].astype(bf16)

    @pl.when((bval_ref[2 * b] > 0) | (bval_ref[2 * b + 1] > 0))
    def _():
        bf = jnp.bfloat16
        wf0b = swf0[...]
        wf1b = swf1[...]
        wc0b = swc0[...]
        wc1b = swc1[...]
        wt0b = swt0[...]
        wt1b = swt1[...]

        for i, (wE0, bE0, wE1, bE1) in enumerate(
                ((we0a, be0a, we1a, be1a), (we0b_, be0b_, we1b_, be1b_))):
            sl = pl.ds(i * BLK, BLK)
            e = bexp_ref[2 * b + i]
            msk8 = lax.broadcasted_iota(jnp.int32, (BLK, NOPT), 1) == e
            x = obs_ref[sl].astype(bf)
            h = jnp.maximum(jnp.dot(x, wf0b, preferred_element_type=jnp.float32) + bf0[...], 0.0)
            st = jnp.dot(h.astype(bf), wf1b, preferred_element_type=jnp.float32) + bf1[...]
            st_b = st.astype(bf)

            hc = jnp.maximum(jnp.dot(st_b, wc0b, preferred_element_type=jnp.float32) + bc0[...], 0.0)
            od = jnp.dot(hc.astype(bf), wc1b, preferred_element_type=jnp.float32) + bc1[...]
            ov = jnp.sum(jnp.where(msk8, od, 0.0), axis=1, keepdims=True)

            ht = jnp.maximum(jnp.dot(st_b, wt0b, preferred_element_type=jnp.float32) + bt0[...], 0.0)
            td = jnp.dot(ht.astype(bf), wt1b, preferred_element_type=jnp.float32) + bt1[...]
            tps = 1.0 / (1.0 + jnp.exp(-td))
            tp = jnp.sum(jnp.where(msk8, tps, 0.0), axis=1, keepdims=True)

            h1 = jnp.maximum(jnp.dot(st_b, wE0[0].astype(bf), preferred_element_type=jnp.float32) + bE0[0], 0.0)
            lg = jnp.dot(h1.astype(bf), wE1[0].astype(bf), preferred_element_type=jnp.float32) + bE1[0]
            mx = jnp.max(lg, axis=1, keepdims=True)
            lse = jnp.log(jnp.sum(jnp.exp(lg - mx), axis=1, keepdims=True)) + mx
            res_ref[sl] = jnp.concatenate(
                [lg, lse, ov, tp, jnp.zeros((BLK, 128 - ACT - 3), jnp.float32)],
                axis=1)


def _const(shape):
    return pl.BlockSpec(shape, lambda b, be, bv, _s=len(shape): (0,) * _s)


def _make_tc(interpret=False):
    grid_spec = pltpu.PrefetchScalarGridSpec(
        num_scalar_prefetch=2,
        grid=(NBLK // 2,),
        in_specs=[
            pl.BlockSpec((2 * BLK, OBS), lambda b, be, bv: (b, 0)),
            _const((OBS, FEAT)),
            _const((1, FEAT)),
            _const((FEAT, FEAT)),
            _const((1, FEAT)),
            _const((FEAT, CRIT_H)),
            _const((1, CRIT_H)),
            _const((CRIT_H, NOPT)),
            _const((1, NOPT)),
            _const((FEAT, TERM_H)),
            _const((1, TERM_H)),
            _const((TERM_H, NOPT)),
            _const((1, NOPT)),
            pl.BlockSpec((1, FEAT, OPT_H), lambda b, be, bv: (be[2 * b], 0, 0)),
            pl.BlockSpec((1, 1, OPT_H), lambda b, be, bv: (be[2 * b], 0, 0)),
            pl.BlockSpec((1, OPT_H, ACT), lambda b, be, bv: (be[2 * b], 0, 0)),
            pl.BlockSpec((1, 1, ACT), lambda b, be, bv: (be[2 * b], 0, 0)),
            pl.BlockSpec((1, FEAT, OPT_H), lambda b, be, bv: (be[2 * b + 1], 0, 0)),
            pl.BlockSpec((1, 1, OPT_H), lambda b, be, bv: (be[2 * b + 1], 0, 0)),
            pl.BlockSpec((1, OPT_H, ACT), lambda b, be, bv: (be[2 * b + 1], 0, 0)),
            pl.BlockSpec((1, 1, ACT), lambda b, be, bv: (be[2 * b + 1], 0, 0)),
        ],
        out_specs=pl.BlockSpec((2 * BLK, 128), lambda b, be, bv: (b, 0)),
        scratch_shapes=[
            pltpu.VMEM((OBS, FEAT), jnp.bfloat16),
            pltpu.VMEM((FEAT, FEAT), jnp.bfloat16),
            pltpu.VMEM((FEAT, CRIT_H), jnp.bfloat16),
            pltpu.VMEM((CRIT_H, NOPT), jnp.bfloat16),
            pltpu.VMEM((FEAT, TERM_H), jnp.bfloat16),
            pltpu.VMEM((TERM_H, NOPT), jnp.bfloat16),
        ],
    )
    return pl.pallas_call(
        _tc_body,
        grid_spec=grid_spec,
        out_shape=jax.ShapeDtypeStruct((PADN, 128), jnp.float32),
        compiler_params=pltpu.CompilerParams(
            dimension_semantics=("arbitrary",)),
        interpret=interpret,
    )


def _gather_body(dst_hbm, act_hbm, res_hbm,
                 lp_out, ov_out, tp_out,
                 dstv, actv, rowbuf, o0, o1, o2, sem):
    nc = 2
    w = lax.axis_index("s") * nc + lax.axis_index("c")
    pltpu.sync_copy(dst_hbm.at[w], dstv)
    pltpu.sync_copy(act_hbm.at[pl.ds(w * TOK_W, TOK_W)], actv)
    pltpu.async_copy(res_hbm.at[dstv], rowbuf, sem).wait()
    for c in range(TOK_W // LANES):
        rows = lax.iota(jnp.int32, LANES) + c * LANES
        av = actv[pl.ds(c * LANES, LANES)]
        la = plsc.load_gather(rowbuf, [rows, av])
        lse = plsc.load_gather(rowbuf, [rows, jnp.full((LANES,), ACT, jnp.int32)])
        o0[pl.ds(c * LANES, LANES)] = la - lse
        o1[pl.ds(c * LANES, LANES)] = plsc.load_gather(
            rowbuf, [rows, jnp.full((LANES,), ACT + 1, jnp.int32)])
        o2[pl.ds(c * LANES, LANES)] = plsc.load_gather(
            rowbuf, [rows, jnp.full((LANES,), ACT + 2, jnp.int32)])
    pltpu.sync_copy(o0, lp_out.at[w])
    pltpu.sync_copy(o1, ov_out.at[w])
    pltpu.sync_copy(o2, tp_out.at[w])


def _make_gather(interpret=False):
    return functools.partial(
        pl.kernel,
        out_type=[jax.ShapeDtypeStruct((NW, TOK_W), jnp.float32)] * 3,
        mesh=plsc.VectorSubcoreMesh(core_axis_name="c", subcore_axis_name="s", num_cores=2, num_subcores=16),
        scratch_types=[
            pltpu.VMEM((TOK_W,), jnp.int32),
            pltpu.VMEM((TOK_W,), jnp.int32),
            pltpu.VMEM((TOK_W, 128), jnp.float32),
            pltpu.VMEM((TOK_W,), jnp.float32),
            pltpu.VMEM((TOK_W,), jnp.float32),
            pltpu.VMEM((TOK_W,), jnp.float32),
            pltpu.SemaphoreType.DMA,
        ],
        compiler_params=pltpu.CompilerParams(needs_layout_passes=False),
        interpret=interpret,
    )(_gather_body)


_route = _make_route()
_tc = _make_tc()
_gather = _make_gather()


def kernel(obs, Wf0, bf0, Wf1, bf1, Wc0, bc0, Wc1, bc1, Wt0, bt0, Wt1, bt1,
           We0, be0, We1, be1, act, opt):
    act32 = act.astype(jnp.int32)
    opt32 = opt.astype(jnp.int32)

    obs_s, dst, bexp, bval = _route(obs, opt32)

    res = _tc(
        bexp, bval,
        obs_s,
        Wf0, bf0.reshape(1, FEAT),
        Wf1, bf1.reshape(1, FEAT),
        Wc0, bc0.reshape(1, CRIT_H),
        Wc1, bc1.reshape(1, NOPT),
        Wt0, bt0.reshape(1, TERM_H),
        Wt1, bt1.reshape(1, NOPT),
        We0, be0.reshape(NOPT, 1, OPT_H),
        We1, be1.reshape(NOPT, 1, ACT),
        We0, be0.reshape(NOPT, 1, OPT_H),
        We1, be1.reshape(NOPT, 1, ACT),
    )

    lp, ov, tp = _gather(dst, act32, res)
    return lp.reshape(N), ov.reshape(N), tp.reshape(N)


# speedup vs baseline: 1.0048x; 1.0048x over previous
import functools

import jax
import jax.numpy as jnp
from jax import lax
from jax.experimental import pallas as pl
from jax.experimental.pallas import tpu as pltpu
from jax.experimental.pallas import tpu_sc as plsc

N = 4096
OBS = 256
FEAT = 1024
CRIT_H = 1024
TERM_H = 512
OPT_H = 1024
NOPT = 8
ACT = 64

BLK = 256
PADN = N + NOPT * BLK
NBLK = PADN // BLK
NBLK_PAD = 32

NW = 32
TOK_W = N // NW
LANES = 16


def _route_body(obs_hbm, opt_hbm,
                obs_s_hbm, dst_hbm, bexp_hbm, bval_hbm,
                optbuf, dstbuf, obsbuf, bexpbuf, bvalbuf, sem):
    nc = 2
    w = lax.axis_index("s") * nc + lax.axis_index("c")

    pltpu.sync_copy(opt_hbm, optbuf)
    obs_cp = pltpu.async_copy(obs_hbm.at[pl.ds(w * TOK_W, TOK_W)], obsbuf, sem)

    myfirst = w * (TOK_W // LANES)
    zero16 = jnp.zeros((LANES,), jnp.int32)

    def group_step(g, accs):
        a4 = zero16
        for k in range(TOK_W // LANES):
            v = optbuf[pl.ds((g * (TOK_W // LANES) + k) * LANES, LANES)]
            a4 = a4 + (jnp.int32(1) << (v * 4))
        return tuple(accs[e] + ((a4 >> (e * 4)) & 15) for e in range(NOPT))

    snap = lax.fori_loop(0, w, group_step, (zero16,) * NOPT)
    accs = lax.fori_loop(w, NW, group_step, snap)

    tot = [jnp.sum(accs[e]) for e in range(NOPT)]
    prefix = [jnp.sum(snap[e]) for e in range(NOPT)]

    base, pad = [], []
    run = jnp.int32(0)
    for e in range(NOPT):
        base.append(run)
        pe = ((tot[e] + (BLK - 1)) // BLK) * BLK
        pad.append(pe)
        run = run + pe

    start = [base[e] + prefix[e] for e in range(NOPT)]
    for c in range(TOK_W // LANES):
        v = optbuf[pl.ds((myfirst + c) * LANES, LANES)]
        dstv = zero16
        for e in range(NOPT):
            m = v == e
            mi = m.astype(jnp.int32)
            ranks = plsc.cumsum(mi)
            dstv = jnp.where(m, start[e] + ranks - 1, dstv)
            start[e] = start[e] + jnp.sum(mi)
        dstbuf[pl.ds(c * LANES, LANES)] = dstv

    pltpu.sync_copy(dstbuf, dst_hbm.at[w])
    obs_cp.wait()
    pltpu.async_copy(obsbuf, obs_s_hbm.at[dstbuf], sem).wait()

    @pl.when(w == 0)
    def _():
        for cc in range(NBLK_PAD // LANES):
            pos = (lax.iota(jnp.int32, LANES) + cc * LANES) * BLK
            ev = jnp.zeros((LANES,), jnp.int32)
            vv = jnp.zeros((LANES,), jnp.int32)
            for e in range(NOPT):
                ev = ev + (pos >= base[e] + pad[e]).astype(jnp.int32)
                vv = vv + ((pos >= base[e]) & (pos < base[e] + tot[e])).astype(jnp.int32)
            bexpbuf[pl.ds(cc * LANES, LANES)] = jnp.minimum(ev, NOPT - 1)
            bvalbuf[pl.ds(cc * LANES, LANES)] = vv
        pltpu.sync_copy(bexpbuf, bexp_hbm)
        pltpu.sync_copy(bvalbuf, bval_hbm)


def _make_route(interpret=False):
    return functools.partial(
        pl.kernel,
        out_type=[
            jax.ShapeDtypeStruct((PADN, OBS), jnp.float32),
            jax.ShapeDtypeStruct((NW, TOK_W), jnp.int32),
            jax.ShapeDtypeStruct((NBLK_PAD,), jnp.int32),
            jax.ShapeDtypeStruct((NBLK_PAD,), jnp.int32),
        ],
        mesh=plsc.VectorSubcoreMesh(core_axis_name="c", subcore_axis_name="s", num_cores=2, num_subcores=16),
        scratch_types=[
            pltpu.VMEM((N,), jnp.int32),
            pltpu.VMEM((TOK_W,), jnp.int32),
            pltpu.VMEM((TOK_W, OBS), jnp.float32),
            pltpu.VMEM((NBLK_PAD,), jnp.int32),
            pltpu.VMEM((NBLK_PAD,), jnp.int32),
            pltpu.SemaphoreType.DMA,
        ],
        compiler_params=pltpu.CompilerParams(needs_layout_passes=False),
        interpret=interpret,
    )(_route_body)


def _tc_body(bexp_ref, bval_ref,
             obs_ref, wf0, bf0, wf1, bf1, wc0, bc0, wc1, bc1,
             wt0, bt0, wt1, bt1, we0a, be0a, we1a, be1a,
             we0b_, be0b_, we1b_, be1b_,
             res_ref,
             swf0, swf1, swc0, swc1, swt0, swt1):
    b = pl.program_id(0)
    bf16 = jnp.bfloat16

    @pl.when(b == 0)
    def _():
        swf0[...] = wf0[...].astype(bf16)
        swf1[...] = wf1[...].astype(bf16)
        swc0[...] = wc0[...].astype(bf16)
        swc1[...] = wc1[...].astype(bf16)
        swt0[...] = wt0[...].astype(bf16)
        swt1[...] = wt1[...].astype(bf16)

    @pl.when((bval_ref[2 * b] > 0) | (bval_ref[2 * b + 1] > 0))
    def _():
        bf = jnp.bfloat16
        wf0b = swf0[...]
        wf1b = swf1[...]
        wc0b = swc0[...]
        wc1b = swc1[...]
        wt0b = swt0[...]
        wt1b = swt1[...]

        for i, (wE0, bE0, wE1, bE1) in enumerate(
                ((we0a, be0a, we1a, be1a), (we0b_, be0b_, we1b_, be1b_))):
            sl = pl.ds(i * BLK, BLK)
            e = bexp_ref[2 * b + i]
            msk8 = lax.broadcasted_iota(jnp.int32, (BLK, NOPT), 1) == e
            x = obs_ref[sl].astype(bf)
            h = jnp.maximum(jnp.dot(x, wf0b, preferred_element_type=jnp.float32) + bf0[...], 0.0)
            st = jnp.dot(h.astype(bf), wf1b, preferred_element_type=jnp.float32) + bf1[...]
            st_b = st.astype(bf)

            hc = jnp.maximum(jnp.dot(st_b, wc0b, preferred_element_type=jnp.float32) + bc0[...], 0.0)
            od = jnp.dot(hc.astype(bf), wc1b, preferred_element_type=jnp.float32) + bc1[...]
            ov = jnp.sum(jnp.where(msk8, od, 0.0), axis=1, keepdims=True)

            ht = jnp.maximum(jnp.dot(st_b, wt0b, preferred_element_type=jnp.float32) + bt0[...], 0.0)
            td = jnp.dot(ht.astype(bf), wt1b, preferred_element_type=jnp.float32) + bt1[...]
            tps = 1.0 / (1.0 + jnp.exp(-td))
            tp = jnp.sum(jnp.where(msk8, tps, 0.0), axis=1, keepdims=True)

            h1 = jnp.maximum(jnp.dot(st_b, wE0[0].astype(bf), preferred_element_type=jnp.float32) + bE0[0], 0.0)
            lg = jnp.dot(h1.astype(bf), wE1[0].astype(bf), preferred_element_type=jnp.float32) + bE1[0]
            mx = jnp.max(lg, axis=1, keepdims=True)
            lse = jnp.log(jnp.sum(jnp.exp(lg - mx), axis=1, keepdims=True)) + mx
            res_ref[sl] = jnp.concatenate(
                [lg, lse, ov, tp, jnp.zeros((BLK, 128 - ACT - 3), jnp.float32)],
                axis=1)


def _const(shape):
    return pl.BlockSpec(shape, lambda b, be, bv, _s=len(shape): (0,) * _s)


def _make_tc(interpret=False):
    grid_spec = pltpu.PrefetchScalarGridSpec(
        num_scalar_prefetch=2,
        grid=(NBLK // 2,),
        in_specs=[
            pl.BlockSpec((2 * BLK, OBS), lambda b, be, bv: (b, 0)),
            _const((OBS, FEAT)),
            _const((1, FEAT)),
            _const((FEAT, FEAT)),
            _const((1, FEAT)),
            _const((FEAT, CRIT_H)),
            _const((1, CRIT_H)),
            _const((CRIT_H, NOPT)),
            _const((1, NOPT)),
            _const((FEAT, TERM_H)),
            _const((1, TERM_H)),
            _const((TERM_H, NOPT)),
            _const((1, NOPT)),
            pl.BlockSpec((1, FEAT, OPT_H), lambda b, be, bv: (be[2 * b], 0, 0)),
            pl.BlockSpec((1, 1, OPT_H), lambda b, be, bv: (be[2 * b], 0, 0)),
            pl.BlockSpec((1, OPT_H, ACT), lambda b, be, bv: (be[2 * b], 0, 0)),
            pl.BlockSpec((1, 1, ACT), lambda b, be, bv: (be[2 * b], 0, 0)),
            pl.BlockSpec((1, FEAT, OPT_H), lambda b, be, bv: (be[2 * b + 1], 0, 0)),
            pl.BlockSpec((1, 1, OPT_H), lambda b, be, bv: (be[2 * b + 1], 0, 0)),
            pl.BlockSpec((1, OPT_H, ACT), lambda b, be, bv: (be[2 * b + 1], 0, 0)),
            pl.BlockSpec((1, 1, ACT), lambda b, be, bv: (be[2 * b + 1], 0, 0)),
        ],
        out_specs=pl.BlockSpec((2 * BLK, 128), lambda b, be, bv: (b, 0)),
        scratch_shapes=[
            pltpu.VMEM((OBS, FEAT), jnp.bfloat16),
            pltpu.VMEM((FEAT, FEAT), jnp.bfloat16),
            pltpu.VMEM((FEAT, CRIT_H), jnp.bfloat16),
            pltpu.VMEM((CRIT_H, NOPT), jnp.bfloat16),
            pltpu.VMEM((FEAT, TERM_H), jnp.bfloat16),
            pltpu.VMEM((TERM_H, NOPT), jnp.bfloat16),
        ],
    )
    return pl.pallas_call(
        _tc_body,
        grid_spec=grid_spec,
        out_shape=jax.ShapeDtypeStruct((PADN, 128), jnp.float32),
        compiler_params=pltpu.CompilerParams(
            dimension_semantics=("arbitrary",)),
        interpret=interpret,
    )


def _gather_body(dst_hbm, act_hbm, res_hbm,
                 lp_out, ov_out, tp_out,
                 dstv, actv, rowbuf, o0, o1, o2, sem):
    nc = 2
    w = lax.axis_index("s") * nc + lax.axis_index("c")
    pltpu.sync_copy(dst_hbm.at[w], dstv)
    pltpu.sync_copy(act_hbm.at[pl.ds(w * TOK_W, TOK_W)], actv)
    pltpu.async_copy(res_hbm.at[dstv], rowbuf, sem).wait()
    for c in range(TOK_W // LANES):
        rows = lax.iota(jnp.int32, LANES) + c * LANES
        av = actv[pl.ds(c * LANES, LANES)]
        la = plsc.load_gather(rowbuf, [rows, av])
        lse = plsc.load_gather(rowbuf, [rows, jnp.full((LANES,), ACT, jnp.int32)])
        o0[pl.ds(c * LANES, LANES)] = la - lse
        o1[pl.ds(c * LANES, LANES)] = plsc.load_gather(
            rowbuf, [rows, jnp.full((LANES,), ACT + 1, jnp.int32)])
        o2[pl.ds(c * LANES, LANES)] = plsc.load_gather(
            rowbuf, [rows, jnp.full((LANES,), ACT + 2, jnp.int32)])
    pltpu.sync_copy(o0, lp_out.at[w])
    pltpu.sync_copy(o1, ov_out.at[w])
    pltpu.sync_copy(o2, tp_out.at[w])


def _make_gather(interpret=False):
    return functools.partial(
        pl.kernel,
        out_type=[jax.ShapeDtypeStruct((NW, TOK_W), jnp.float32)] * 3,
        mesh=plsc.VectorSubcoreMesh(core_axis_name="c", subcore_axis_name="s", num_cores=2, num_subcores=16),
        scratch_types=[
            pltpu.VMEM((TOK_W,), jnp.int32),
            pltpu.VMEM((TOK_W,), jnp.int32),
            pltpu.VMEM((TOK_W, 128), jnp.float32),
            pltpu.VMEM((TOK_W,), jnp.float32),
            pltpu.VMEM((TOK_W,), jnp.float32),
            pltpu.VMEM((TOK_W,), jnp.float32),
            pltpu.SemaphoreType.DMA,
        ],
        compiler_params=pltpu.CompilerParams(needs_layout_passes=False),
        interpret=interpret,
    )(_gather_body)


_route = _make_route()
_tc = _make_tc()
_gather = _make_gather()


def kernel(obs, Wf0, bf0, Wf1, bf1, Wc0, bc0, Wc1, bc1, Wt0, bt0, Wt1, bt1,
           We0, be0, We1, be1, act, opt):
    act32 = act.astype(jnp.int32)
    opt32 = opt.astype(jnp.int32)

    obs_s, dst, bexp, bval = _route(obs, opt32)

    res = _tc(
        bexp, bval,
        obs_s,
        Wf0, bf0.reshape(1, FEAT),
        Wf1, bf1.reshape(1, FEAT),
        Wc0, bc0.reshape(1, CRIT_H),
        Wc1, bc1.reshape(1, NOPT),
        Wt0, bt0.reshape(1, TERM_H),
        Wt1, bt1.reshape(1, NOPT),
        We0, be0.reshape(NOPT, 1, OPT_H),
        We1, be1.reshape(NOPT, 1, ACT),
        We0, be0.reshape(NOPT, 1, OPT_H),
        We1, be1.reshape(NOPT, 1, ACT),
    )

    lp, ov, tp = _gather(dst, act32, res)
    return lp.reshape(N), ov.reshape(N), tp.reshape(N)


# speedup vs baseline: 1.0103x; 1.0054x over previous
import functools

import jax
import jax.numpy as jnp
from jax import lax
from jax.experimental import pallas as pl
from jax.experimental.pallas import tpu as pltpu
from jax.experimental.pallas import tpu_sc as plsc

N = 4096
OBS = 256
FEAT = 1024
CRIT_H = 1024
TERM_H = 512
OPT_H = 1024
NOPT = 8
ACT = 64

BLK = 256
PADN = N + NOPT * BLK
NBLK = PADN // BLK
NBLK_PAD = 32

NW = 32
TOK_W = N // NW
LANES = 16


def _route_body(obs_hbm, opt_hbm,
                obs_s_hbm, dst_hbm, bexp_hbm, bval_hbm,
                optbuf, dstbuf, obsbuf, bexpbuf, bvalbuf, sem):
    nc = 2
    w = lax.axis_index("s") * nc + lax.axis_index("c")

    pltpu.sync_copy(opt_hbm, optbuf)
    obs_cp = pltpu.async_copy(obs_hbm.at[pl.ds(w * TOK_W, TOK_W)], obsbuf, sem)

    myfirst = w * (TOK_W // LANES)
    zero16 = jnp.zeros((LANES,), jnp.int32)

    def group_step(g, accs):
        a4 = zero16
        for k in range(TOK_W // LANES):
            v = optbuf[pl.ds((g * (TOK_W // LANES) + k) * LANES, LANES)]
            a4 = a4 + (jnp.int32(1) << (v * 4))
        return tuple(accs[e] + ((a4 >> (e * 4)) & 15) for e in range(NOPT))

    snap = lax.fori_loop(0, w, group_step, (zero16,) * NOPT)
    accs = lax.fori_loop(w, NW, group_step, snap)

    tot = [jnp.sum(accs[e]) for e in range(NOPT)]
    prefix = [jnp.sum(snap[e]) for e in range(NOPT)]

    base, pad = [], []
    run = jnp.int32(0)
    for e in range(NOPT):
        base.append(run)
        pe = ((tot[e] + (BLK - 1)) // BLK) * BLK
        pad.append(pe)
        run = run + pe

    start = [base[e] + prefix[e] for e in range(NOPT)]
    for c in range(TOK_W // LANES):
        v = optbuf[pl.ds((myfirst + c) * LANES, LANES)]
        dstv = zero16
        for e in range(NOPT):
            m = v == e
            mi = m.astype(jnp.int32)
            ranks = plsc.cumsum(mi)
            dstv = jnp.where(m, start[e] + ranks - 1, dstv)
            start[e] = start[e] + jnp.sum(mi)
        dstbuf[pl.ds(c * LANES, LANES)] = dstv

    pltpu.sync_copy(dstbuf, dst_hbm.at[w])
    obs_cp.wait()
    pltpu.async_copy(obsbuf, obs_s_hbm.at[dstbuf], sem).wait()

    @pl.when(w == 0)
    def _():
        for cc in range(NBLK_PAD // LANES):
            pos = (lax.iota(jnp.int32, LANES) + cc * LANES) * BLK
            ev = jnp.zeros((LANES,), jnp.int32)
            vv = jnp.zeros((LANES,), jnp.int32)
            for e in range(NOPT):
                ev = ev + (pos >= base[e] + pad[e]).astype(jnp.int32)
                vv = vv + ((pos >= base[e]) & (pos < base[e] + tot[e])).astype(jnp.int32)
            bexpbuf[pl.ds(cc * LANES, LANES)] = jnp.minimum(ev, NOPT - 1)
            bvalbuf[pl.ds(cc * LANES, LANES)] = vv
        pltpu.sync_copy(bexpbuf, bexp_hbm)
        pltpu.sync_copy(bvalbuf, bval_hbm)


def _make_route(interpret=False):
    return functools.partial(
        pl.kernel,
        out_type=[
            jax.ShapeDtypeStruct((PADN, OBS), jnp.float32),
            jax.ShapeDtypeStruct((NW, TOK_W), jnp.int32),
            jax.ShapeDtypeStruct((NBLK_PAD,), jnp.int32),
            jax.ShapeDtypeStruct((NBLK_PAD,), jnp.int32),
        ],
        mesh=plsc.VectorSubcoreMesh(core_axis_name="c", subcore_axis_name="s", num_cores=2, num_subcores=16),
        scratch_types=[
            pltpu.VMEM((N,), jnp.int32),
            pltpu.VMEM((TOK_W,), jnp.int32),
            pltpu.VMEM((TOK_W, OBS), jnp.float32),
            pltpu.VMEM((NBLK_PAD,), jnp.int32),
            pltpu.VMEM((NBLK_PAD,), jnp.int32),
            pltpu.SemaphoreType.DMA,
        ],
        compiler_params=pltpu.CompilerParams(needs_layout_passes=False),
        interpret=interpret,
    )(_route_body)


def _tc_body(bexp_ref, bval_ref,
             obs_ref, wf0, bf0, wf1, bf1, wc0, bc0, wc1, bc1,
             wt0, bt0, wt1, bt1, we0a, be0a, we1a, be1a,
             we0b_, be0b_, we1b_, be1b_,
             res_ref,
             swf0, swf1, swc0, swc1, swt0, swt1):
    b = pl.program_id(0)
    bf16 = jnp.bfloat16

    @pl.when(b == 0)
    def _():
        swf0[...] = wf0[...].astype(bf16)
        swf1[...] = wf1[...].astype(bf16)
        swc0[...] = wc0[...].astype(bf16)
        swc1[...] = wc1[...].astype(bf16)
        swt0[...] = wt0[...].astype(bf16)
        swt1[...] = wt1[...].astype(bf16)

    @pl.when((bval_ref[2 * b] > 0) | (bval_ref[2 * b + 1] > 0))
    def _():
        bf = jnp.bfloat16
        wf0b = swf0[...]
        wf1b = swf1[...]
        wc0b = swc0[...]
        wc1b = swc1[...]
        wt0b = swt0[...]
        wt1b = swt1[...]

        for i, (wE0, bE0, wE1, bE1) in enumerate(
                ((we0a, be0a, we1a, be1a), (we0b_, be0b_, we1b_, be1b_))):
            sl = pl.ds(i * BLK, BLK)
            e = bexp_ref[2 * b + i]
            msk8 = lax.broadcasted_iota(jnp.int32, (BLK, NOPT), 1) == e
            x = obs_ref[sl].astype(bf)
            h = jnp.maximum(jnp.dot(x, wf0b, preferred_element_type=jnp.float32) + bf0[...], 0.0)
            st = jnp.dot(h.astype(bf), wf1b, preferred_element_type=jnp.float32) + bf1[...]
            st_b = st.astype(bf)

            hc = jnp.maximum(jnp.dot(st_b, wc0b, preferred_element_type=jnp.float32) + bc0[...], 0.0)
            od = jnp.dot(hc.astype(bf), wc1b, preferred_element_type=jnp.float32) + bc1[...]
            ov = jnp.sum(jnp.where(msk8, od, 0.0), axis=1, keepdims=True)

            ht = jnp.maximum(jnp.dot(st_b, wt0b, preferred_element_type=jnp.float32) + bt0[...], 0.0)
            td = jnp.dot(ht.astype(bf), wt1b, preferred_element_type=jnp.float32) + bt1[...]
            tps = 1.0 / (1.0 + jnp.exp(-td))
            tp = jnp.sum(jnp.where(msk8, tps, 0.0), axis=1, keepdims=True)

            h1 = jnp.maximum(jnp.dot(st_b, wE0[0].astype(bf), preferred_element_type=jnp.float32) + bE0[0], 0.0)
            lg = jnp.dot(h1.astype(bf), wE1[0].astype(bf), preferred_element_type=jnp.float32) + bE1[0]
            mx = jnp.max(lg, axis=1, keepdims=True)
            lse = jnp.log(jnp.sum(jnp.exp(lg - mx), axis=1, keepdims=True)) + mx
            res_ref[sl] = jnp.concatenate(
                [lg, lse, ov, tp, jnp.zeros((BLK, 128 - ACT - 3), jnp.float32)],
                axis=1)


def _const(shape):
    return pl.BlockSpec(shape, lambda b, be, bv, _s=len(shape): (0,) * _s)


def _make_tc(interpret=False):
    grid_spec = pltpu.PrefetchScalarGridSpec(
        num_scalar_prefetch=2,
        grid=(NBLK // 2,),
        in_specs=[
            pl.BlockSpec((2 * BLK, OBS), lambda b, be, bv: (b, 0)),
            _const((OBS, FEAT)),
            _const((1, FEAT)),
            _const((FEAT, FEAT)),
            _const((1, FEAT)),
            _const((FEAT, CRIT_H)),
            _const((1, CRIT_H)),
            _const((CRIT_H, NOPT)),
            _const((1, NOPT)),
            _const((FEAT, TERM_H)),
            _const((1, TERM_H)),
            _const((TERM_H, NOPT)),
            _const((1, NOPT)),
            pl.BlockSpec((1, FEAT, OPT_H), lambda b, be, bv: (be[2 * b], 0, 0)),
            pl.BlockSpec((1, 1, OPT_H), lambda b, be, bv: (be[2 * b], 0, 0)),
            pl.BlockSpec((1, OPT_H, ACT), lambda b, be, bv: (be[2 * b], 0, 0)),
            pl.BlockSpec((1, 1, ACT), lambda b, be, bv: (be[2 * b], 0, 0)),
            pl.BlockSpec((1, FEAT, OPT_H), lambda b, be, bv: (be[2 * b + 1], 0, 0)),
            pl.BlockSpec((1, 1, OPT_H), lambda b, be, bv: (be[2 * b + 1], 0, 0)),
            pl.BlockSpec((1, OPT_H, ACT), lambda b, be, bv: (be[2 * b + 1], 0, 0)),
            pl.BlockSpec((1, 1, ACT), lambda b, be, bv: (be[2 * b + 1], 0, 0)),
        ],
        out_specs=pl.BlockSpec((2 * BLK, 128), lambda b, be, bv: (b, 0)),
        scratch_shapes=[
            pltpu.VMEM((OBS, FEAT), jnp.bfloat16),
            pltpu.VMEM((FEAT, FEAT), jnp.bfloat16),
            pltpu.VMEM((FEAT, CRIT_H), jnp.bfloat16),
            pltpu.VMEM((CRIT_H, NOPT), jnp.bfloat16),
            pltpu.VMEM((FEAT, TERM_H), jnp.bfloat16),
            pltpu.VMEM((TERM_H, NOPT), jnp.bfloat16),
        ],
    )
    return pl.pallas_call(
        _tc_body,
        grid_spec=grid_spec,
        out_shape=jax.ShapeDtypeStruct((PADN, 128), jnp.float32),
        compiler_params=pltpu.CompilerParams(
            dimension_semantics=("arbitrary",)),
        interpret=interpret,
    )


def _gather_body(dst_hbm, act_hbm, res_hbm,
                 lp_out, ov_out, tp_out,
                 dstv, actv, rowbuf, o0, o1, o2, sem, sem2):
    nc = 2
    w = lax.axis_index("s") * nc + lax.axis_index("c")
    act_cp = pltpu.async_copy(act_hbm.at[pl.ds(w * TOK_W, TOK_W)], actv, sem2)
    pltpu.sync_copy(dst_hbm.at[w], dstv)
    pltpu.async_copy(res_hbm.at[dstv], rowbuf, sem).wait()
    act_cp.wait()
    for c in range(TOK_W // LANES):
        rows = lax.iota(jnp.int32, LANES) + c * LANES
        av = actv[pl.ds(c * LANES, LANES)]
        la = plsc.load_gather(rowbuf, [rows, av])
        lse = plsc.load_gather(rowbuf, [rows, jnp.full((LANES,), ACT, jnp.int32)])
        o0[pl.ds(c * LANES, LANES)] = la - lse
        o1[pl.ds(c * LANES, LANES)] = plsc.load_gather(
            rowbuf, [rows, jnp.full((LANES,), ACT + 1, jnp.int32)])
        o2[pl.ds(c * LANES, LANES)] = plsc.load_gather(
            rowbuf, [rows, jnp.full((LANES,), ACT + 2, jnp.int32)])
    pltpu.sync_copy(o0, lp_out.at[w])
    pltpu.sync_copy(o1, ov_out.at[w])
    pltpu.sync_copy(o2, tp_out.at[w])


def _make_gather(interpret=False):
    return functools.partial(
        pl.kernel,
        out_type=[jax.ShapeDtypeStruct((NW, TOK_W), jnp.float32)] * 3,
        mesh=plsc.VectorSubcoreMesh(core_axis_name="c", subcore_axis_name="s", num_cores=2, num_subcores=16),
        scratch_types=[
            pltpu.VMEM((TOK_W,), jnp.int32),
            pltpu.VMEM((TOK_W,), jnp.int32),
            pltpu.VMEM((TOK_W, 128), jnp.float32),
            pltpu.VMEM((TOK_W,), jnp.float32),
            pltpu.VMEM((TOK_W,), jnp.float32),
            pltpu.VMEM((TOK_W,), jnp.float32),
            pltpu.SemaphoreType.DMA,
            pltpu.SemaphoreType.DMA,
        ],
        compiler_params=pltpu.CompilerParams(needs_layout_passes=False),
        interpret=interpret,
    )(_gather_body)


_route = _make_route()
_tc = _make_tc()
_gather = _make_gather()


def kernel(obs, Wf0, bf0, Wf1, bf1, Wc0, bc0, Wc1, bc1, Wt0, bt0, Wt1, bt1,
           We0, be0, We1, be1, act, opt):
    act32 = act.astype(jnp.int32)
    opt32 = opt.astype(jnp.int32)

    obs_s, dst, bexp, bval = _route(obs, opt32)

    res = _tc(
        bexp, bval,
        obs_s,
        Wf0, bf0.reshape(1, FEAT),
        Wf1, bf1.reshape(1, FEAT),
        Wc0, bc0.reshape(1, CRIT_H),
        Wc1, bc1.reshape(1, NOPT),
        Wt0, bt0.reshape(1, TERM_H),
        Wt1, bt1.reshape(1, NOPT),
        We0, be0.reshape(NOPT, 1, OPT_H),
        We1, be1.reshape(NOPT, 1, ACT),
        We0, be0.reshape(NOPT, 1, OPT_H),
        We1, be1.reshape(NOPT, 1, ACT),
    )

    lp, ov, tp = _gather(dst, act32, res)
    return lp.reshape(N), ov.reshape(N), tp.reshape(N)
